# baseline (speedup 1.0000x reference)
import jax
import jax.numpy as jnp
from jax.experimental import pallas as pl
from jax.experimental.pallas import tpu as pltpu

N = 512
NC2 = N * (N - 1) // 2


def _tri_body(flat_ref, o_ref, m_ref):
    lane = jax.lax.broadcasted_iota(jnp.int32, (N,), 0)

    def row(i, carry):
        start = (i * (i - 1)) // 2
        base = pl.multiple_of((start // 128) * 128, 128)
        sh = start % 128
        r = flat_ref[0, :, pl.ds(base, N + 128)]
        r = pltpu.roll(r, ((N + 128) - sh) % (N + 128), 1)
        m_ref[i, :] = jnp.where(lane < i, r[0, :N], 0.0)
        return carry

    jax.lax.fori_loop(0, N, row, 0)
    m = m_ref[...]
    o_ref[0] = m + m.T


def _tri_call(flat_padded, interpret=False):
    b = flat_padded.shape[0]
    return pl.pallas_call(
        _tri_body,
        grid=(b,),
        in_specs=[pl.BlockSpec((1, 1, NC2 + N), lambda i: (i, 0, 0))],
        out_specs=pl.BlockSpec((1, N, N), lambda i: (i, 0, 0)),
        out_shape=jax.ShapeDtypeStruct((b, N, N), jnp.float32),
        scratch_shapes=[pltpu.VMEM((N, N), jnp.float32)],
        interpret=interpret,
    )(flat_padded)


def kernel(decompFE):
    flat_padded = jnp.pad(decompFE, ((0, 0), (0, N)))
    return _tri_call(flat_padded.reshape(-1, 1, NC2 + N))


# speedup vs baseline: 9.8837x; 9.8837x over previous
import functools

import jax
import jax.numpy as jnp
from jax import lax
from jax.experimental import pallas as pl
from jax.experimental.pallas import tpu as pltpu
from jax.experimental.pallas import tpu_sc as plsc

N = 512
NC2 = N * (N - 1) // 2
B = 128

SC_CORES = 2
SC_SUBCORES = 16
NW = SC_CORES * SC_SUBCORES
BATCH_PER_W = B // NW

RB = 32
NBLK = N // RB

def _tri(i):
    return (i * (i - 1)) // 2

_A = []
_L = []
for _k in range(NBLK):
    a = (_tri(RB * _k) // 8) * 8
    end = _tri(RB * (_k + 1))
    l = -(-(end - a) // 8) * 8
    _A.append(a)
    _L.append(l)
CHUNK_MAX = max(_L) + N + 16


def _sc_build_body(flat_hbm, p_hbm, chunk_v, strip_v):
    wid = lax.axis_index("s") * SC_CORES + lax.axis_index("c")
    lane = lax.iota(jnp.int32, 16)

    def per_batch(bb, carry):
        b = wid * BATCH_PER_W + bb
        for k in range(NBLK):
            src_off = pl.multiple_of(b * NC2 + _A[k], 8)
            pltpu.sync_copy(flat_hbm.at[pl.ds(src_off, _L[k])],
                            chunk_v.at[pl.ds(0, _L[k])])
            w_k = RB * (k + 1)

            def per_row(r, c2, k=k, w_k=w_k):
                i = RB * k + r
                off = (i * (i - 1)) // 2 - _A[k]
                for g in range(w_k // 16):
                    idx = off + g * 16 + lane
                    v = plsc.load_gather(chunk_v, [idx])
                    strip_v[r, pl.ds(g * 16, 16)] = v
                return c2

            lax.fori_loop(0, RB, per_row, 0)
            pltpu.sync_copy(strip_v, p_hbm.at[b, pl.ds(RB * k, RB)])
        return carry

    lax.fori_loop(0, BATCH_PER_W, per_batch, 0)


@functools.cache
def _sc_build():
    return pl.kernel(
        _sc_build_body,
        mesh=plsc.VectorSubcoreMesh(core_axis_name="c", subcore_axis_name="s"),
        out_type=jax.ShapeDtypeStruct((B, N, N), jnp.float32),
        scratch_types=[
            pltpu.VMEM((CHUNK_MAX,), jnp.float32),
            pltpu.VMEM((RB, N), jnp.float32),
        ],
        compiler_params=pltpu.CompilerParams(needs_layout_passes=False),
    )


STRIP = 128
NSTRIP = N // STRIP


def _sym_body(rows_ref, cols_ref, o_ref):
    s = pl.program_id(1)
    i_glob = jax.lax.broadcasted_iota(jnp.int32, (STRIP, N), 0) + s * STRIP
    j = jax.lax.broadcasted_iota(jnp.int32, (STRIP, N), 1)
    low = jnp.where(j < i_glob, rows_ref[0], 0.0)
    a = jax.lax.broadcasted_iota(jnp.int32, (N, STRIP), 0)
    c = jax.lax.broadcasted_iota(jnp.int32, (N, STRIP), 1) + s * STRIP
    up = jnp.where(c < a, cols_ref[0], 0.0).T
    o_ref[0] = low + up


def _sym_call(p, interpret=False):
    b = p.shape[0]
    return pl.pallas_call(
        _sym_body,
        grid=(b, NSTRIP),
        in_specs=[
            pl.BlockSpec((1, STRIP, N), lambda i, s: (i, s, 0)),
            pl.BlockSpec((1, N, STRIP), lambda i, s: (i, 0, s)),
        ],
        out_specs=pl.BlockSpec((1, STRIP, N), lambda i, s: (i, s, 0)),
        out_shape=jax.ShapeDtypeStruct((b, N, N), jnp.float32),
        interpret=interpret,
    )(p, p)


def kernel(decompFE):
    p = _sc_build()(decompFE.reshape(-1))
    return _sym_call(p)
